# baseline (speedup 1.0000x reference)
import functools

import jax
import jax.numpy as jnp
from jax import lax
from jax.experimental import pallas as pl
from jax.experimental.pallas import tpu as pltpu
from jax.experimental.pallas import tpu_sc as plsc

_K = 32
_BIG = 3.0e38


def _sc_topk(xs, ys, B, N, K):
    NW = 32
    RPW = (B * N) // NW
    WPB = N // RPW
    NCH = N // 16
    CAP = N + 64
    mesh = plsc.VectorSubcoreMesh(core_axis_name="c", subcore_axis_name="s")

    @functools.partial(
        pl.kernel,
        mesh=mesh,
        compiler_params=pltpu.CompilerParams(needs_layout_passes=False),
        out_type=[
            jax.ShapeDtypeStruct((B * N * K,), jnp.float32),
            jax.ShapeDtypeStruct((B * N * K,), jnp.int32),
        ],
        scratch_types=[
            pltpu.VMEM((N,), jnp.float32),
            pltpu.VMEM((N,), jnp.float32),
            pltpu.VMEM((N,), jnp.float32),
            pltpu.VMEM((N,), jnp.float32),
            pltpu.VMEM((CAP,), jnp.float32),
            pltpu.VMEM((CAP,), jnp.int32),
            pltpu.VMEM((CAP,), jnp.float32),
            pltpu.VMEM((CAP,), jnp.int32),
            pltpu.VMEM((RPW * K,), jnp.float32),
            pltpu.VMEM((RPW * K,), jnp.int32),
        ],
    )
    def topk(xs_hbm, ys_hbm, vout_hbm, iout_hbm,
             xs_v, ys_v, db0, db1, cd0, ci0, cd1, ci1, ov, oi):
        wid = lax.axis_index("c") * 16 + lax.axis_index("s")
        batch = wid // WPB
        row0 = (wid % WPB) * RPW
        big = jnp.float32(_BIG)
        big16 = jnp.full((16,), big, jnp.float32)
        z16 = jnp.zeros((16,), jnp.int32)
        iota = lax.iota(jnp.int32, 16)

        pltpu.sync_copy(xs_hbm.at[batch], xs_v)
        pltpu.sync_copy(ys_hbm.at[batch], ys_v)

        def pair_body(pr, _):
            r0 = 2 * pr
            i0 = row0 + r0
            i0_16 = jnp.broadcast_to(i0, (16,))
            i1_16 = i0_16 + 1
            x0 = plsc.load_gather(xs_v, [i0_16])
            y0 = plsc.load_gather(ys_v, [i0_16])
            x1 = plsc.load_gather(xs_v, [i1_16])
            y1 = plsc.load_gather(ys_v, [i1_16])

            def dist2(c):
                o = c * 16
                xj = xs_v[pl.ds(o, 16)]
                yj = ys_v[pl.ds(o, 16)]
                colv = iota + o
                dx0 = xj - x0
                dy0 = yj - y0
                d0 = dx0 * dx0 + dy0 * dy0
                d0 = jnp.where(colv == i0_16, big, d0)
                db0[pl.ds(o, 16)] = d0
                dx1 = xj - x1
                dy1 = yj - y1
                d1 = dx1 * dx1 + dy1 * dy1
                d1 = jnp.where(colv == i1_16, big, d1)
                db1[pl.ds(o, 16)] = d1
                return d0, d1

            @plsc.parallel_loop(jnp.int32(0), jnp.int32(NCH // 2),
                                jnp.int32(1),
                                carry=(big16, big16, big16, big16), unroll=4)
            def pa(t, accs):
                t = lax.convert_element_type(t, jnp.int32)
                ae0, ao0, ae1, ao1 = accs
                de0, de1 = dist2(2 * t)
                do0, do1 = dist2(2 * t + 1)
                return (jnp.minimum(ae0, de0), jnp.minimum(ao0, do0),
                        jnp.minimum(ae1, de1), jnp.minimum(ao1, do1))

            ae0, ao0, ae1, ao1 = pa
            if NCH % 2:
                dl0, dl1 = dist2(NCH - 1)
                ae0 = jnp.minimum(ae0, dl0)
                ae1 = jnp.minimum(ae1, dl1)
            T0 = jnp.max(jnp.maximum(ae0, ao0))
            T1 = jnp.max(jnp.maximum(ae1, ao1))
            T0_16 = jnp.broadcast_to(T0, (16,))
            T1_16 = jnp.broadcast_to(T1, (16,))

            @plsc.parallel_loop(jnp.int32(0), jnp.int32(NCH), jnp.int32(1),
                                carry=(z16, z16), unroll=4)
            def pb(c, cnts):
                cnt0, cnt1 = cnts
                o = lax.convert_element_type(c, jnp.int32) * 16
                colv = iota + o
                d0 = db0[pl.ds(o, 16)]
                m0 = d0 <= T0_16
                cs0 = plsc.cumsum(m0.astype(jnp.int32))
                plsc.store_scatter(cd0, [cnt0 + cs0 - 1], d0, mask=m0)
                plsc.store_scatter(ci0, [cnt0 + cs0 - 1], colv, mask=m0)
                d1 = db1[pl.ds(o, 16)]
                m1 = d1 <= T1_16
                cs1 = plsc.cumsum(m1.astype(jnp.int32))
                plsc.store_scatter(cd1, [cnt1 + cs1 - 1], d1, mask=m1)
                plsc.store_scatter(ci1, [cnt1 + cs1 - 1], colv, mask=m1)
                return (cnt0 + plsc.all_reduce_population_count(m0),
                        cnt1 + plsc.all_reduce_population_count(m1))

            cnt0_16, cnt1_16 = pb
            cnt0 = jnp.max(cnt0_16)
            cnt1 = jnp.max(cnt1_16)
            plsc.store_scatter(cd0, [cnt0_16 + iota], big16)
            plsc.store_scatter(cd1, [cnt1_16 + iota], big16)

            def merge(lo, lov, hi, hiv, ck, cv):
                rk = lax.rev(ck, (0,))
                rv = lax.rev(cv, (0,))
                m = hi <= rk
                nhk = jnp.where(m, hi, rk)
                nhv = jnp.where(m, hiv, rv)
                m2 = lo <= nhk
                ak = jnp.where(m2, lo, nhk)
                av = jnp.where(m2, lov, nhv)
                bk = jnp.where(m2, nhk, lo)
                bv = jnp.where(m2, nhv, lov)
                lo, lov = plsc.sort_key_val(ak, av)
                hi, hiv = plsc.sort_key_val(bk, bv)
                return lo, lov, hi, hiv

            nv0 = (cnt0 + 15) // 16
            nv1 = (cnt1 + 15) // 16
            nvm = jnp.maximum(nv0, nv1)
            nv0_16 = jnp.broadcast_to(nv0, (16,))
            nv1_16 = jnp.broadcast_to(nv1, (16,))

            def pc(j, st):
                l0, lv0, h0, hv0, l1, lv1, h1, hv1 = st
                o = j * 16
                j16 = jnp.broadcast_to(j, (16,))
                ck0 = jnp.where(j16 < nv0_16, cd0[pl.ds(o, 16)], big16)
                ck1 = jnp.where(j16 < nv1_16, cd1[pl.ds(o, 16)], big16)
                sk0, sv0 = plsc.sort_key_val(ck0, ci0[pl.ds(o, 16)])
                sk1, sv1 = plsc.sort_key_val(ck1, ci1[pl.ds(o, 16)])
                l0, lv0, h0, hv0 = merge(l0, lv0, h0, hv0, sk0, sv0)
                l1, lv1, h1, hv1 = merge(l1, lv1, h1, hv1, sk1, sv1)
                return l0, lv0, h0, hv0, l1, lv1, h1, hv1

            l0, lv0, h0, hv0, l1, lv1, h1, hv1 = lax.fori_loop(
                jnp.int32(0), nvm, pc,
                (big16, z16, big16, z16, big16, z16, big16, z16))

            off = jnp.broadcast_to(batch * N, (16,))
            ov[pl.ds(r0 * K, 16)] = l0
            ov[pl.ds(r0 * K + 16, 16)] = h0
            oi[pl.ds(r0 * K, 16)] = lv0 + off
            oi[pl.ds(r0 * K + 16, 16)] = hv0 + off
            ov[pl.ds(r0 * K + 32, 16)] = l1
            ov[pl.ds(r0 * K + 48, 16)] = h1
            oi[pl.ds(r0 * K + 32, 16)] = lv1 + off
            oi[pl.ds(r0 * K + 48, 16)] = hv1 + off
            return _

        lax.fori_loop(jnp.int32(0), jnp.int32(RPW // 2), pair_body, 0)
        pltpu.sync_copy(ov, vout_hbm.at[pl.ds(wid * RPW * K, RPW * K)])
        pltpu.sync_copy(oi, iout_hbm.at[pl.ds(wid * RPW * K, RPW * K)])

    return topk(xs, ys)


def _edge_body(v_ref, w_ref, b_ref, out_ref, src_ref, *, rows, emb):
    gi = pl.program_id(0)
    sv = jnp.sqrt(v_ref[...])
    svt = sv.T
    for j in range(rows):
        out_ref[pl.ds(j * emb, emb), :] = (
            svt[:, j:j + 1] * w_ref[...] + b_ref[...])
    r = gi * rows + jax.lax.broadcasted_iota(jnp.int32, (rows, emb), 0)
    lane = jax.lax.broadcasted_iota(jnp.int32, (rows, emb), 1)
    e = r * emb + lane
    src_ref[...] = jax.lax.shift_right_logical(e, jnp.int32(5))


def kernel(locs, init_embeddings, W, b):
    B, N, _ = locs.shape
    EMB = W.shape[0]
    K = _K

    xs = locs[..., 0]
    ys = locs[..., 1]
    valsq, dst = _sc_topk(xs, ys, B, N, K)

    ROWS = 400
    vals_2d = valsq.reshape((B * N * K) // EMB, EMB)
    w_row = W.reshape(1, EMB)
    b_row = b.reshape(1, EMB)
    GE = (B * N * K) // (ROWS * EMB)
    edge_emb, src = pl.pallas_call(
        functools.partial(_edge_body, rows=ROWS, emb=EMB),
        grid=(GE,),
        in_specs=[
            pl.BlockSpec((ROWS, EMB), lambda gi: (gi, gi * 0)),
            pl.BlockSpec((1, EMB), lambda gi: (gi * 0, gi * 0)),
            pl.BlockSpec((1, EMB), lambda gi: (gi * 0, gi * 0)),
        ],
        out_specs=[
            pl.BlockSpec((ROWS * EMB, EMB), lambda gi: (gi, gi * 0)),
            pl.BlockSpec((ROWS, EMB), lambda gi: (gi, gi * 0)),
        ],
        out_shape=[
            jax.ShapeDtypeStruct((B * N * K, EMB), jnp.float32),
            jax.ShapeDtypeStruct(((B * N * K) // EMB, EMB), jnp.int32),
        ],
    )(vals_2d, w_row, b_row)

    edge_index = jnp.stack(
        [src.reshape(-1), dst]).astype(jnp.int64)
    x = init_embeddings.reshape(B * N, EMB)
    return x, edge_index, edge_emb


# speedup vs baseline: 1.0043x; 1.0043x over previous
import functools

import jax
import jax.numpy as jnp
from jax import lax
from jax.experimental import pallas as pl
from jax.experimental.pallas import tpu as pltpu
from jax.experimental.pallas import tpu_sc as plsc

_K = 32
_BIG = 3.0e38


def _sc_topk(xs, ys, B, N, K):
    NW = 32
    RPW = (B * N) // NW
    WPB = N // RPW
    NCH = N // 16
    CAP = N + 64
    mesh = plsc.VectorSubcoreMesh(core_axis_name="c", subcore_axis_name="s")

    @functools.partial(
        pl.kernel,
        mesh=mesh,
        compiler_params=pltpu.CompilerParams(needs_layout_passes=False),
        out_type=[
            jax.ShapeDtypeStruct((B * N * K,), jnp.float32),
            jax.ShapeDtypeStruct((B * N * K,), jnp.int32),
        ],
        scratch_types=[
            pltpu.VMEM((N,), jnp.float32),
            pltpu.VMEM((N,), jnp.float32),
            pltpu.VMEM((N,), jnp.float32),
            pltpu.VMEM((N,), jnp.float32),
            pltpu.VMEM((CAP,), jnp.float32),
            pltpu.VMEM((CAP,), jnp.int32),
            pltpu.VMEM((CAP,), jnp.float32),
            pltpu.VMEM((CAP,), jnp.int32),
            pltpu.VMEM((RPW * K,), jnp.float32),
            pltpu.VMEM((RPW * K,), jnp.int32),
        ],
    )
    def topk(xs_hbm, ys_hbm, vout_hbm, iout_hbm,
             xs_v, ys_v, db0, db1, cd0, ci0, cd1, ci1, ov, oi):
        wid = lax.axis_index("c") * 16 + lax.axis_index("s")
        batch = wid // WPB
        row0 = (wid % WPB) * RPW
        big = jnp.float32(_BIG)
        big16 = jnp.full((16,), big, jnp.float32)
        z16 = jnp.zeros((16,), jnp.int32)
        iota = lax.iota(jnp.int32, 16)

        pltpu.sync_copy(xs_hbm.at[batch], xs_v)
        pltpu.sync_copy(ys_hbm.at[batch], ys_v)

        def pair_body(pr, _):
            r0 = 2 * pr
            i0 = row0 + r0
            i0_16 = jnp.broadcast_to(i0, (16,))
            i1_16 = i0_16 + 1
            x0 = plsc.load_gather(xs_v, [i0_16])
            y0 = plsc.load_gather(ys_v, [i0_16])
            x1 = plsc.load_gather(xs_v, [i1_16])
            y1 = plsc.load_gather(ys_v, [i1_16])

            def dist2(c):
                o = c * 16
                xj = xs_v[pl.ds(o, 16)]
                yj = ys_v[pl.ds(o, 16)]
                colv = iota + o
                dx0 = xj - x0
                dy0 = yj - y0
                d0 = dx0 * dx0 + dy0 * dy0
                d0 = jnp.where(colv == i0_16, big, d0)
                db0[pl.ds(o, 16)] = d0
                dx1 = xj - x1
                dy1 = yj - y1
                d1 = dx1 * dx1 + dy1 * dy1
                d1 = jnp.where(colv == i1_16, big, d1)
                db1[pl.ds(o, 16)] = d1
                return d0, d1

            @plsc.parallel_loop(jnp.int32(0), jnp.int32(NCH // 2),
                                jnp.int32(1),
                                carry=(big16, big16, big16, big16), unroll=4)
            def pa(t, accs):
                t = lax.convert_element_type(t, jnp.int32)
                ae0, ao0, ae1, ao1 = accs
                de0, de1 = dist2(2 * t)
                do0, do1 = dist2(2 * t + 1)
                return (jnp.minimum(ae0, de0), jnp.minimum(ao0, do0),
                        jnp.minimum(ae1, de1), jnp.minimum(ao1, do1))

            ae0, ao0, ae1, ao1 = pa
            if NCH % 2:
                dl0, dl1 = dist2(NCH - 1)
                ae0 = jnp.minimum(ae0, dl0)
                ae1 = jnp.minimum(ae1, dl1)
            T0 = jnp.max(jnp.maximum(ae0, ao0))
            T1 = jnp.max(jnp.maximum(ae1, ao1))
            T0_16 = jnp.broadcast_to(T0, (16,))
            T1_16 = jnp.broadcast_to(T1, (16,))

            @plsc.parallel_loop(jnp.int32(0), jnp.int32(NCH), jnp.int32(1),
                                carry=(z16, z16), unroll=4)
            def pb(c, cnts):
                cnt0, cnt1 = cnts
                o = lax.convert_element_type(c, jnp.int32) * 16
                colv = iota + o
                d0 = db0[pl.ds(o, 16)]
                m0 = d0 <= T0_16
                cs0 = plsc.cumsum(m0.astype(jnp.int32))
                plsc.store_scatter(cd0, [cnt0 + cs0 - 1], d0, mask=m0)
                plsc.store_scatter(ci0, [cnt0 + cs0 - 1], colv, mask=m0)
                d1 = db1[pl.ds(o, 16)]
                m1 = d1 <= T1_16
                cs1 = plsc.cumsum(m1.astype(jnp.int32))
                plsc.store_scatter(cd1, [cnt1 + cs1 - 1], d1, mask=m1)
                plsc.store_scatter(ci1, [cnt1 + cs1 - 1], colv, mask=m1)
                return (cnt0 + plsc.all_reduce_population_count(m0),
                        cnt1 + plsc.all_reduce_population_count(m1))

            cnt0_16, cnt1_16 = pb
            cnt0 = jnp.max(cnt0_16)
            cnt1 = jnp.max(cnt1_16)
            plsc.store_scatter(cd0, [cnt0_16 + iota], big16)
            plsc.store_scatter(cd1, [cnt1_16 + iota], big16)

            def merge(lo, lov, hi, hiv, ck, cv):
                rk = lax.rev(ck, (0,))
                rv = lax.rev(cv, (0,))
                m = hi <= rk
                nhk = jnp.where(m, hi, rk)
                nhv = jnp.where(m, hiv, rv)
                m2 = lo <= nhk
                ak = jnp.where(m2, lo, nhk)
                av = jnp.where(m2, lov, nhv)
                bk = jnp.where(m2, nhk, lo)
                bv = jnp.where(m2, nhv, lov)
                lo, lov = plsc.sort_key_val(ak, av)
                hi, hiv = plsc.sort_key_val(bk, bv)
                return lo, lov, hi, hiv

            nv0 = (cnt0 + 15) // 16
            nv1 = (cnt1 + 15) // 16
            nvm = jnp.maximum(nv0, nv1)
            nv0_16 = jnp.broadcast_to(nv0, (16,))
            nv1_16 = jnp.broadcast_to(nv1, (16,))

            def pc(j, st):
                l0, lv0, h0, hv0, l1, lv1, h1, hv1 = st
                o = j * 16
                j16 = jnp.broadcast_to(j, (16,))
                ck0 = jnp.where(j16 < nv0_16, cd0[pl.ds(o, 16)], big16)
                ck1 = jnp.where(j16 < nv1_16, cd1[pl.ds(o, 16)], big16)
                sk0, sv0 = plsc.sort_key_val(ck0, ci0[pl.ds(o, 16)])
                sk1, sv1 = plsc.sort_key_val(ck1, ci1[pl.ds(o, 16)])
                l0, lv0, h0, hv0 = merge(l0, lv0, h0, hv0, sk0, sv0)
                l1, lv1, h1, hv1 = merge(l1, lv1, h1, hv1, sk1, sv1)
                return l0, lv0, h0, hv0, l1, lv1, h1, hv1

            l0, lv0, h0, hv0, l1, lv1, h1, hv1 = lax.fori_loop(
                jnp.int32(0), nvm, pc,
                (big16, z16, big16, z16, big16, z16, big16, z16))

            off = jnp.broadcast_to(batch * N, (16,))
            ov[pl.ds(r0 * K, 16)] = l0
            ov[pl.ds(r0 * K + 16, 16)] = h0
            oi[pl.ds(r0 * K, 16)] = lv0 + off
            oi[pl.ds(r0 * K + 16, 16)] = hv0 + off
            ov[pl.ds(r0 * K + 32, 16)] = l1
            ov[pl.ds(r0 * K + 48, 16)] = h1
            oi[pl.ds(r0 * K + 32, 16)] = lv1 + off
            oi[pl.ds(r0 * K + 48, 16)] = hv1 + off
            return _

        lax.fori_loop(jnp.int32(0), jnp.int32(RPW // 2), pair_body, 0)
        pltpu.sync_copy(ov, vout_hbm.at[pl.ds(wid * RPW * K, RPW * K)])
        pltpu.sync_copy(oi, iout_hbm.at[pl.ds(wid * RPW * K, RPW * K)])

    return topk(xs, ys)


def _edge_body(v_ref, w_ref, b_ref, out_ref, src_ref, *, rows, emb):
    gi = pl.program_id(0)
    sv = jnp.sqrt(v_ref[...])
    svt = sv.T
    for j in range(rows):
        out_ref[pl.ds(j * emb, emb), :] = (
            svt[:, j:j + 1] * w_ref[...] + b_ref[...])
    r = gi * rows + jax.lax.broadcasted_iota(jnp.int32, (rows, emb), 0)
    lane = jax.lax.broadcasted_iota(jnp.int32, (rows, emb), 1)
    e = r * emb + lane
    src_ref[...] = jax.lax.shift_right_logical(e, jnp.int32(5))


def kernel(locs, init_embeddings, W, b):
    B, N, _ = locs.shape
    EMB = W.shape[0]
    K = _K

    xs = locs[..., 0]
    ys = locs[..., 1]
    valsq, dst = _sc_topk(xs, ys, B, N, K)

    ROWS = 200
    vals_2d = valsq.reshape((B * N * K) // EMB, EMB)
    w_row = W.reshape(1, EMB)
    b_row = b.reshape(1, EMB)
    GE = (B * N * K) // (ROWS * EMB)
    edge_emb, src = pl.pallas_call(
        functools.partial(_edge_body, rows=ROWS, emb=EMB),
        grid=(GE,),
        in_specs=[
            pl.BlockSpec((ROWS, EMB), lambda gi: (gi, gi * 0)),
            pl.BlockSpec((1, EMB), lambda gi: (gi * 0, gi * 0)),
            pl.BlockSpec((1, EMB), lambda gi: (gi * 0, gi * 0)),
        ],
        out_specs=[
            pl.BlockSpec((ROWS * EMB, EMB), lambda gi: (gi, gi * 0)),
            pl.BlockSpec((ROWS, EMB), lambda gi: (gi, gi * 0)),
        ],
        out_shape=[
            jax.ShapeDtypeStruct((B * N * K, EMB), jnp.float32),
            jax.ShapeDtypeStruct(((B * N * K) // EMB, EMB), jnp.int32),
        ],
    )(vals_2d, w_row, b_row)

    edge_index = jnp.stack(
        [src.reshape(-1), dst]).astype(jnp.int64)
    x = init_embeddings.reshape(B * N, EMB)
    return x, edge_index, edge_emb
